# baseline (speedup 1.0000x reference)
import functools

import jax
import jax.numpy as jnp
from jax import lax
from jax.experimental import pallas as pl
from jax.experimental.pallas import tpu as pltpu
from jax.experimental.pallas import tpu_sc as plsc

B, N, M, D = 4, 4096, 4096, 3
NCORE, NSUB, L = 2, 16, 16
NW = NCORE * NSUB
WPB = NW // B
RPW = N // WPB
NCH = M // L

_INF = jnp.float32(jnp.inf)


def _sc_body(pred_hbm, gt_hbm, outp_hbm, d2stage_hbm,
             ploc, gloc, d2loc, mbuf, stage, tr):
    c = lax.axis_index("c")
    s = lax.axis_index("s")
    wid = c * NSUB + s
    grp = s // WPB
    b = c * 2 + grp
    r = s % WPB

    pltpu.sync_copy(pred_hbm.at[pl.ds(b * WPB + r, 1)], ploc)
    pltpu.sync_copy(gt_hbm.at[pl.ds(b, 1)], gloc)

    def init_j(j, _):
        d2loc[0, pl.ds(j * L, L)] = jnp.full((L,), _INF, jnp.float32)
        return 0

    lax.fori_loop(0, NCH, init_j, 0)

    iot = lax.iota(jnp.int32, L)
    oh = (jnp.int32(1) - jnp.minimum(iot, jnp.int32(1))).astype(jnp.float32)
    negbase = (oh - 1.0) * jnp.float32(3e38)
    tr[0, pl.ds(L, L)] = jnp.full((L,), _INF, jnp.float32)

    def blk_body(rb, carry):
        sumv, maxv = carry
        base = rb * L
        pxv = ploc[0, pl.ds(base, L)]
        pyv = ploc[0, pl.ds(RPW + base, L)]
        pzv = ploc[0, pl.ds(2 * RPW + base, L)]
        px = [pxv[k] for k in range(L)]
        py = [pyv[k] for k in range(L)]
        pz = [pzv[k] for k in range(L)]

        def ch_body(j, raccs):
            off = j * L
            gx = gloc[0, pl.ds(off, L)]
            gy = gloc[0, pl.ds(M + off, L)]
            gz = gloc[0, pl.ds(2 * M + off, L)]
            d2cur = d2loc[0, pl.ds(off, L)]
            out = []
            for k in range(L):
                dx = px[k] - gx
                dy = py[k] - gy
                dz = pz[k] - gz
                d = dx * dx + dy * dy + dz * dz
                d2cur = jnp.minimum(d2cur, d)
                out.append(jnp.minimum(raccs[k], d))
            d2loc[0, pl.ds(off, L)] = d2cur
            return tuple(out)

        raccs = lax.fori_loop(
            0, NCH, ch_body,
            tuple(jnp.full((L,), _INF, jnp.float32) for _ in range(L)))

        for k in range(L):
            v = raccs[k]
            for sh in (8, 4, 2, 1):
                tr[0, pl.ds(0, L)] = v
                v = jnp.minimum(v, tr[0, pl.ds(sh, L)])
            sumv = sumv + v * oh
            maxv = jnp.maximum(maxv, v * oh + negbase)

        return sumv, maxv

    sumv, maxv = lax.fori_loop(
        0, RPW // L, blk_body,
        (jnp.zeros((L,), jnp.float32),
         jnp.full((L,), -3e38, jnp.float32)))

    pltpu.sync_copy(d2loc, d2stage_hbm.at[pl.ds(wid, 1)])
    plsc.subcore_barrier()

    base_w = c * NSUB + grp * WPB
    for k in range(WPB):
        pltpu.sync_copy(d2stage_hbm.at[pl.ds(base_w + k, 1)],
                        mbuf.at[pl.ds(k, 1)])

    def m_body(j, acc):
        off = j * L
        v = mbuf[0, pl.ds(off, L)]
        for k in range(1, WPB):
            v = jnp.minimum(v, mbuf[k, pl.ds(off, L)])
        return acc + v

    sum2v = lax.fori_loop(0, NCH, m_body, jnp.zeros((L,), jnp.float32))

    stage[0, :] = sumv
    pltpu.sync_copy(stage, outp_hbm.at[pl.ds(3 * wid, 1)])
    stage[0, :] = maxv
    pltpu.sync_copy(stage, outp_hbm.at[pl.ds(3 * wid + 1, 1)])
    stage[0, :] = sum2v
    pltpu.sync_copy(stage, outp_hbm.at[pl.ds(3 * wid + 2, 1)])


@jax.jit
def _sc_call(pred_r, gt_t):
    mesh = plsc.VectorSubcoreMesh(core_axis_name="c", subcore_axis_name="s")
    fn = functools.partial(
        pl.kernel,
        mesh=mesh,
        out_type=[
            jax.ShapeDtypeStruct((3 * NW, L), jnp.float32),
            jax.ShapeDtypeStruct((NW, M), jnp.float32),
        ],
        scratch_types=[
            pltpu.VMEM((1, D * RPW), jnp.float32),
            pltpu.VMEM((1, D * M), jnp.float32),
            pltpu.VMEM((1, M), jnp.float32),
            pltpu.VMEM((WPB, M), jnp.float32),
            pltpu.VMEM((1, L), jnp.float32),
            pltpu.VMEM((1, 2 * L), jnp.float32),
        ],
    )(_sc_body)
    return fn(pred_r, gt_t)


def kernel(pred, gt):
    pred_r = jnp.transpose(pred, (0, 2, 1)).reshape(B, D, WPB, RPW)
    pred_r = jnp.transpose(pred_r, (0, 2, 1, 3)).reshape(B * WPB, D * RPW)
    gt_t = jnp.transpose(gt, (0, 2, 1)).reshape(B, D * M)

    outp, _ = _sc_call(pred_r, gt_t)

    parts = outp.reshape(2, 2, WPB, 3, L)
    sum1_b = jnp.sum(parts[:, :, :, 0, :], axis=(2, 3))
    max1_b = jnp.max(parts[:, :, :, 1, :], axis=(2, 3))
    sum2_b = jnp.sum(parts[:, :, 0, 2, :], axis=2)
    loss_b = sum1_b / N + sum2_b / M + max1_b
    return jnp.mean(loss_b)


# speedup vs baseline: 8.6262x; 8.6262x over previous
import jax
import jax.numpy as jnp
from jax.experimental import pallas as pl
from jax.experimental.pallas import tpu as pltpu

B, N, M = 4, 4096, 4096
BLK_N = 1024
NB = N // BLK_N
BLK_M = 1024
NC = M // BLK_M
LANES = 128
SUB = 8


def _chamfer_body(p_ref, g_ref, out_ref, dist2_ref, acc_ref):
    b = pl.program_id(0)
    i = pl.program_id(1)

    p = p_ref[0]
    x2 = jnp.sum(p * p, axis=1, keepdims=True)

    gxyz = g_ref[0]
    y2 = jnp.sum(gxyz * gxyz, axis=0, keepdims=True)
    gs = gxyz * -2.0

    rowpart = None
    for j in range(NC):
        g = gs[:, j * BLK_M:(j + 1) * BLK_M]
        c = jax.lax.dot_general(
            p, g, (((1,), (0,)), ((), ())),
            preferred_element_type=jnp.float32,
        )
        s = x2 + y2[:, j * BLK_M:(j + 1) * BLK_M]
        d = s + c

        part = d[:, 0:LANES]
        for k in range(1, BLK_M // LANES):
            part = jnp.minimum(part, d[:, k * LANES:(k + 1) * LANES])
        rowpart = part if rowpart is None else jnp.minimum(rowpart, part)

        cpart = d[0:SUB, :]
        for k in range(1, BLK_N // SUB):
            cpart = jnp.minimum(cpart, d[k * SUB:(k + 1) * SUB, :])

        sl = slice(j * BLK_M, (j + 1) * BLK_M)

        @pl.when(i == 0)
        def _():
            dist2_ref[:, sl] = cpart

        @pl.when(i > 0)
        def _():
            dist2_ref[:, sl] = jnp.minimum(dist2_ref[:, sl], cpart)

    rowmin = jnp.min(rowpart, axis=1)
    bsum = jnp.sum(rowmin)
    bmax = jnp.max(rowmin)

    @pl.when(i == 0)
    def _():
        acc_ref[0] = bsum
        acc_ref[1] = bmax

    @pl.when(i > 0)
    def _():
        acc_ref[0] = acc_ref[0] + bsum
        acc_ref[1] = jnp.maximum(acc_ref[1], bmax)

    @pl.when(jnp.logical_and(b == 0, i == 0))
    def _():
        out_ref[0, 0] = 0.0

    @pl.when(i == NB - 1)
    def _():
        mean1 = acc_ref[0] / N
        max1 = acc_ref[1]
        mean2 = jnp.sum(jnp.min(dist2_ref[...], axis=0)) / M
        out_ref[0, 0] = out_ref[0, 0] + (mean1 + mean2 + max1) / B


def kernel(pred, gt):
    gt_t = jnp.transpose(gt, (0, 2, 1))

    out = pl.pallas_call(
        _chamfer_body,
        grid=(B, NB),
        in_specs=[
            pl.BlockSpec((1, BLK_N, 3), lambda b, i: (b, i, 0)),
            pl.BlockSpec((1, 3, M), lambda b, i: (b, 0, 0)),
        ],
        out_specs=pl.BlockSpec(
            (1, 1), lambda b, i: (0, 0), memory_space=pltpu.SMEM
        ),
        out_shape=jax.ShapeDtypeStruct((1, 1), jnp.float32),
        scratch_shapes=[
            pltpu.VMEM((SUB, M), jnp.float32),
            pltpu.SMEM((2,), jnp.float32),
        ],
    )(pred, gt_t)
    return out[0, 0]
